# Initial kernel scaffold; baseline (speedup 1.0000x reference)
import jax
import jax.numpy as jnp
from jax.experimental import pallas as pl


def kernel(edge_index, s_fixed, s, v, wdfp, s_batch, params):
    raise NotImplementedError("write your pallas kernel here")


# speedup vs baseline: 1.0192x; 1.0192x over previous
import jax
import jax.numpy as jnp
import numpy as np
from jax.experimental import pallas as pl

N, E, T, N_GRAPHS = 50000, 800000, 2, 16


def _norm(x, axis=-1, keepdims=False, eps=1e-8):
    return jnp.sqrt(jnp.clip(jnp.sum(x * x, axis=axis, keepdims=keepdims), eps, None))


def _gvp(p, s, v, scalar_act, vector_act):
    vt = jnp.swapaxes(v, -1, -2)
    vh = vt @ p["wh"]
    vn = _norm(vh, axis=-2)
    s_out = jnp.concatenate([s, vn], axis=-1) @ p["ws"] + p["bs"]
    v_out = None
    if "wv" in p:
        v_out = jnp.swapaxes(vh @ p["wv"], -1, -2)
        if vector_act is not None:
            v_out = v_out * vector_act(_norm(v_out, axis=-1, keepdims=True))
    if scalar_act is not None:
        s_out = scalar_act(s_out)
    return s_out, v_out


def kernel(edge_index, s_fixed, s, v, wdfp, s_batch, params):
    relu, sig = jax.nn.relu, jax.nn.sigmoid
    src, dst = edge_index[0], edge_index[1]
    s_h, v_h, wdfp_h = s[:, 0], v[:, 0], wdfp[:, 0]
    out_labels, out_s, out_v = [], [], []
    label_losses = 0.0
    feat_losses = 0.0
    for i in range(1, T + 1):
        si = jnp.concatenate([s_fixed, s_h, wdfp_h], axis=-1)
        vi = v_h
        for lp in params["layers"]:
            se, ve = _gvp(lp["n_encode"], si, vi, relu, sig)
            sm, vm = _gvp(lp["m_gvp"],
                          jnp.concatenate([se[dst], se[src]], axis=1),
                          jnp.concatenate([ve[dst], ve[src]], axis=1), relu, sig)
            s_agg = jax.ops.segment_sum(sm, dst, num_segments=N)
            v_agg = jax.ops.segment_sum(vm, dst, num_segments=N)
            si, vi = _gvp(lp["u_gvp"],
                          jnp.concatenate([se, s_agg], axis=1),
                          jnp.concatenate([ve, v_agg], axis=1), relu, sig)
        lab_s, _ = _gvp(params["label_gvp"], si, vi, relu, sig)
        wdfp_h = lab_s @ params["ln_w"] + params["ln_b"]
        _, v_fp = _gvp(params["feat_pred"], si, vi, None, None)
        s_h = _norm(v_fp)
        v_h = v_fp / s_h[..., None]
        out_labels.append(wdfp_h)
        out_s.append(s_h)
        out_v.append(v_h)
        label_losses = label_losses + jnp.sum(jnp.abs(wdfp_h - wdfp[:, i])) / (N_GRAPHS * 1)
        l1s = jnp.sum(np.e ** (s_h - 2.0) * jnp.abs(s_h - s[:, i])) / (N_GRAPHS * 16)
        l1v = jnp.sum(np.e ** (v_h - 2.0) * jnp.abs(v_h - v[:, i])) / (N_GRAPHS * 16 * 3)
        feat_losses = feat_losses + (l1s + l1v) / 2.0
    loss = (label_losses + feat_losses) / T
    return jnp.stack(out_labels, axis=1), jnp.stack(out_s, axis=1), jnp.stack(out_v, axis=1), loss


# speedup vs baseline: 20.3937x; 20.0096x over previous
import functools

import jax
import jax.numpy as jnp
from jax import lax
from jax.experimental import pallas as pl
from jax.experimental.pallas import tpu as pltpu, tpu_sc as plsc

EPS = 1e-8


def _mm(a, b):
    return jax.lax.dot_general(a.astype(jnp.bfloat16), b.astype(jnp.bfloat16),
                               (((a.ndim - 1,), (0,)), ((), ())),
                               preferred_element_type=jnp.float32)

def _bd3(w):
    z = jnp.zeros_like(w)
    return jnp.concatenate([
        jnp.concatenate([w, z, z], 1),
        jnp.concatenate([z, w, z], 1),
        jnp.concatenate([z, z, w], 1)], 0)


def _norm3(a, b, c):
    return jnp.sqrt(jnp.clip(a * a + b * b + c * c, EPS, None))


def _node_pre_body(siA_ref, siB_ref, wh_ref, wv_ref, ws_ref, bs_ref, out_ref):
    siA = siA_ref[...]
    si = siA[:, :32]
    v48 = jnp.concatenate([siA[:, 32:40], siB_ref[...]], 1)
    hv = _mm(v48, _bd3(wh_ref[...]))
    vh = [hv[:, 16 * a:16 * a + 16] for a in range(3)]
    ov = _mm(hv, _bd3(wv_ref[...]))
    vo = [ov[:, 16 * a:16 * a + 16] for a in range(3)]
    vn = _norm3(*vh)
    se = _mm(jnp.concatenate([si, vn], 1), ws_ref[...]) + bs_ref[...]
    se = jnp.maximum(se, 0.0)
    sc = jax.nn.sigmoid(_norm3(*vo))
    pad = jnp.zeros((se.shape[0], 48), jnp.float32)
    out_ref[...] = jnp.concatenate(
        [se, vo[0] * sc, vo[1] * sc, vo[2] * sc, pad], 1)


def _pair_gvp_body(split_out, xd_ref, xs_ref,
                   wh_ref, wv_ref, ws_ref, bs_ref, *outs):
    xd, xs = xd_ref[...], xs_ref[...]
    vd = xd[:, 32:80]
    vs = xs[:, 32:80]
    vcat = jnp.concatenate(
        [jnp.concatenate([vd[:, 16 * a:16 * a + 16],
                          vs[:, 16 * a:16 * a + 16]], 1) for a in range(3)], 1)
    hv = _mm(vcat, _bd3(wh_ref[...]))
    vh = [hv[:, 32 * a:32 * a + 32] for a in range(3)]
    ov = _mm(hv, _bd3(wv_ref[...]))
    vo = [ov[:, 16 * a:16 * a + 16] for a in range(3)]
    vn = _norm3(*vh)
    sm = _mm(jnp.concatenate([xd[:, :32], xs[:, :32], vn], 1), ws_ref[...]) \
        + bs_ref[...]
    sm = jnp.maximum(sm, 0.0)
    sc = jax.nn.sigmoid(_norm3(*vo))
    vm = [vo[a] * sc for a in range(3)]
    if split_out:
        o1_ref, o2_ref = outs
        o1_ref[...] = jnp.concatenate([sm, vm[0][:, :8]], 1)
        o2_ref[...] = jnp.concatenate([vm[0][:, 8:], vm[1], vm[2]], 1)
    else:
        pad = jnp.zeros((sm.shape[0], 48), jnp.float32)
        outs[0][...] = jnp.concatenate([sm, vm[0], vm[1], vm[2], pad], 1)


def _head_body(stA_ref, stB_ref, wdfp_ref, s_ref, v_ref,
               whl_ref, wsl_ref, bsl_ref, lnw_ref, lnb_ref, whf_ref, wvf_ref,
               lab_ref, sh_ref, vhA_ref, vhB_ref, lp_ref):
    stA, stB = stA_ref[...], stB_ref[...]
    si = stA[:, :32]
    v48 = jnp.concatenate([stA[:, 32:40], stB], 1)
    hvl = _mm(v48, _bd3(whl_ref[...]))
    vnl = _norm3(hvl[:, :16], hvl[:, 16:32], hvl[:, 32:48])
    lab = _mm(jnp.concatenate([si, vnl], 1), wsl_ref[...]) + bsl_ref[...]
    lab = jnp.maximum(lab, 0.0)
    wd = _mm(lab, lnw_ref[...]) + lnb_ref[...]
    vfp = _mm(_mm(v48, _bd3(whf_ref[...])), _bd3(wvf_ref[...]))
    sh = _norm3(vfp[:, :16], vfp[:, 16:32], vfp[:, 32:48])
    sh3 = jnp.concatenate([sh, sh, sh], 1)
    vh = vfp / sh3
    lab_ref[...] = wd
    sh_ref[...] = sh
    vhA_ref[...] = vh[:, :8]
    vhB_ref[...] = vh[:, 8:]
    lt = jnp.sum(jnp.abs(wd - wdfp_ref[...]))
    st = jnp.sum(jnp.exp(sh - 2.0) * jnp.abs(sh - s_ref[...]))
    vt = jnp.sum(jnp.exp(vh - 2.0) * jnp.abs(vh - v_ref[...]))
    part = jnp.concatenate([lt.reshape(1, 1), st.reshape(1, 1),
                            vt.reshape(1, 1)], 1)

    @pl.when(pl.program_id(0) == 0)
    def _():
        lp_ref[...] = jnp.zeros_like(lp_ref)

    lp_ref[...] += part


def _full(shape2):
    return pl.BlockSpec(shape2, lambda i: tuple(0 for _ in shape2))


def _node_pre(siA, siB, p):
    n = siA.shape[0]
    bn = 5000 if n % 5000 == 0 else n
    return pl.pallas_call(
        _node_pre_body,
        grid=(n // bn,),
        in_specs=[pl.BlockSpec((bn, 40), lambda i: (i, 0)),
                  pl.BlockSpec((bn, 40), lambda i: (i, 0)),
                  _full((16, 16)), _full((16, 16)), _full((48, 32)),
                  _full((1, 32))],
        out_specs=pl.BlockSpec((bn, 128), lambda i: (i, 0)),
        out_shape=jax.ShapeDtypeStruct((n, 128), jnp.float32),
    )(siA, siB, p["wh"], p["wv"], p["ws"], p["bs"].reshape(1, 32))


def _pair_gvp(xd, xs, p, blk, split_out):
    n = xd.shape[0]
    bn = blk if n % blk == 0 else n
    if split_out:
        out_specs = [pl.BlockSpec((bn, 40), lambda i: (i, 0)),
                     pl.BlockSpec((bn, 40), lambda i: (i, 0))]
        out_shape = [jax.ShapeDtypeStruct((n, 40), jnp.float32),
                     jax.ShapeDtypeStruct((n, 40), jnp.float32)]
    else:
        out_specs = pl.BlockSpec((bn, 128), lambda i: (i, 0))
        out_shape = jax.ShapeDtypeStruct((n, 128), jnp.float32)
    return pl.pallas_call(
        functools.partial(_pair_gvp_body, split_out),
        grid=(n // bn,),
        in_specs=[pl.BlockSpec((bn, 128), lambda i: (i, 0)),
                  pl.BlockSpec((bn, 128), lambda i: (i, 0)),
                  _full((32, 32)), _full((32, 16)), _full((96, 32)),
                  _full((1, 32))],
        out_specs=out_specs,
        out_shape=out_shape,
    )(xd, xs, p["wh"], p["wv"], p["ws"], p["bs"].reshape(1, 32))


def _head(stA, stB, wdfp_i, s_i, vt_i, params):
    n = stA.shape[0]
    bn = 2000 if n % 2000 == 0 else n
    pla = params["label_gvp"]
    pf = params["feat_pred"]
    return pl.pallas_call(
        _head_body,
        grid=(n // bn,),
        in_specs=[pl.BlockSpec((bn, 40), lambda i: (i, 0)),
                  pl.BlockSpec((bn, 40), lambda i: (i, 0)),
                  pl.BlockSpec((bn, 1), lambda i: (i, 0)),
                  pl.BlockSpec((bn, 16), lambda i: (i, 0)),
                  pl.BlockSpec((bn, 48), lambda i: (i, 0)),
                  _full((16, 16)), _full((48, 64)), _full((1, 64)),
                  _full((64, 1)), _full((1, 1)),
                  _full((16, 16)), _full((16, 16))],
        out_specs=[pl.BlockSpec((bn, 1), lambda i: (i, 0)),
                   pl.BlockSpec((bn, 16), lambda i: (i, 0)),
                   pl.BlockSpec((bn, 8), lambda i: (i, 0)),
                   pl.BlockSpec((bn, 40), lambda i: (i, 0)),
                   pl.BlockSpec((1, 3), lambda i: (0, 0))],
        out_shape=[jax.ShapeDtypeStruct((n, 1), jnp.float32),
                   jax.ShapeDtypeStruct((n, 16), jnp.float32),
                   jax.ShapeDtypeStruct((n, 8), jnp.float32),
                   jax.ShapeDtypeStruct((n, 40), jnp.float32),
                   jax.ShapeDtypeStruct((1, 3), jnp.float32)],
    )(stA, stB, wdfp_i, s_i, vt_i,
      pla["wh"], pla["ws"], pla["bs"].reshape(1, 64),
      params["ln_w"], params["ln_b"].reshape(1, 1),
      pf["wh"], pf["wv"])


@functools.cache
def _sc_mesh():
    return plsc.VectorSubcoreMesh(core_axis_name="c", subcore_axis_name="s")


_CD = 128


def _sc_gather(table, dst, src):
    e = dst.shape[0]
    per_w = e // 32
    n_main, tail = per_w // _CD, per_w % _CD

    @functools.partial(
        pl.kernel,
        out_type=(jax.ShapeDtypeStruct((e, 128), jnp.float32),
                  jax.ShapeDtypeStruct((e, 128), jnp.float32)),
        mesh=_sc_mesh(),
        scratch_types=[pltpu.VMEM((_CD,), jnp.int32),
                       pltpu.VMEM((_CD,), jnp.int32),
                       pltpu.VMEM((_CD, 128), jnp.float32),
                       pltpu.VMEM((_CD, 128), jnp.float32),
                       pltpu.SemaphoreType.DMA,
                       pltpu.SemaphoreType.DMA],
    )
    def k(tab, d_h, s_h, outd, outs, idxd, idxs, rowd, rows_, semd, sems):
        wid = lax.axis_index("s") * 2 + lax.axis_index("c")
        base = wid * per_w

        def chunk(off, sz):
            sl = pl.ds(0, sz)
            pltpu.sync_copy(d_h.at[pl.ds(off, sz)], idxd.at[sl])
            pltpu.sync_copy(s_h.at[pl.ds(off, sz)], idxs.at[sl])
            cd = pltpu.async_copy(tab.at[idxd.at[sl]], rowd.at[sl], semd)
            cs = pltpu.async_copy(tab.at[idxs.at[sl]], rows_.at[sl], sems)
            cd.wait()
            cs.wait()
            pltpu.sync_copy(rowd.at[sl], outd.at[pl.ds(off, sz)])
            pltpu.sync_copy(rows_.at[sl], outs.at[pl.ds(off, sz)])

        def body(j, carry):
            chunk(base + j * _CD, _CD)
            return carry

        lax.fori_loop(0, n_main, body, 0)
        if tail:
            chunk(base + n_main * _CD, tail)

    return k(table, dst, src)


_R = 12800
_G = 128
_AR = _R + _G
_ZN = _AR // 16


def _sc_scatter(msgs, dst, zeros, n):
    e = msgs.shape[0]
    per_t = e // 16
    n_main, tail = per_t // _CD, per_t % _CD
    dn = 800
    dn3, dl3 = 728, n - 3 * _R - 15 * 728

    @functools.partial(
        pl.kernel,
        out_type=jax.ShapeDtypeStruct((n, 128), jnp.float32),
        mesh=_sc_mesh(),
        scratch_types=[pltpu.VMEM_SHARED((_AR, 128), jnp.float32),
                       pltpu.VMEM((_CD,), jnp.int32),
                       pltpu.VMEM((_CD,), jnp.int32),
                       pltpu.VMEM((_CD, 128), jnp.float32)] +
                      ([pltpu.VMEM((tail,), jnp.int32),
                        pltpu.VMEM((tail,), jnp.int32),
                        pltpu.VMEM((tail, 128), jnp.float32)] if tail else []),
    )
    def k(m_h, d_h, z_h, out, acc, idxv, lidx, valv, *tailbufs):
        c = lax.axis_index("c")
        s = lax.axis_index("s")
        lanes = lax.iota(jnp.int32, 16)

        for phase in range(2):
            rng = 2 * c + phase
            base = rng * _R
            pltpu.sync_copy(z_h.at[pl.ds(0, _ZN)],
                            acc.at[pl.ds(s * _ZN, _ZN)])
            plsc.subcore_barrier()

            def chunk(j, off, sz, ibuf, lbuf, vbuf):
                garb = _R + ((lanes + j * 16) & (_G - 1))
                pltpu.sync_copy(d_h.at[pl.ds(off, sz)], ibuf)
                for i in range(sz // 16):
                    d16 = ibuf[pl.ds(16 * i, 16)]
                    loc = d16 - base
                    ok = (loc >= 0) & (loc < _R)
                    lbuf[pl.ds(16 * i, 16)] = jnp.where(ok, loc, garb)
                pltpu.sync_copy(m_h.at[pl.ds(off, sz)], vbuf)
                pltpu.sync_copy(vbuf, acc.at[lbuf], add=True)

            def body(j, carry):
                chunk(j, s * per_t + j * _CD, _CD, idxv, lidx, valv)
                return carry

            lax.fori_loop(0, n_main, body, 0)
            if tail:
                chunk(n_main, s * per_t + n_main * _CD, tail, *tailbufs)
            plsc.subcore_barrier()

            @pl.when(rng < 3)
            def _():
                pltpu.sync_copy(acc.at[pl.ds(s * dn, dn)],
                                out.at[pl.ds(base + s * dn, dn)])

            @pl.when((rng == 3) & (s < 15))
            def _():
                pltpu.sync_copy(acc.at[pl.ds(s * dn3, dn3)],
                                out.at[pl.ds(base + s * dn3, dn3)])

            @pl.when((rng == 3) & (s == 15))
            def _():
                pltpu.sync_copy(acc.at[pl.ds(15 * dn3, dl3)],
                                out.at[pl.ds(base + 15 * dn3, dl3)])

            plsc.subcore_barrier()

    return k(msgs, dst, zeros)


def kernel(edge_index, s_fixed, s, v, wdfp, s_batch, params):
    n, t = s.shape[0], s.shape[1] - 1
    ng = 16
    src = edge_index[0]
    dst = edge_index[1]
    vt = jnp.swapaxes(v, -1, -2).reshape(n, t + 1, 48)
    zeros = jnp.zeros((_ZN, 128), jnp.float32)

    s_h = s[:, 0]
    wdfp_h = wdfp[:, 0]
    vA = vt[:, 0, :8]
    vB = vt[:, 0, 8:]

    out_labels, out_s, out_v, lparts = [], [], [], []
    for i in range(1, t + 1):
        siA = jnp.concatenate([s_fixed, s_h, wdfp_h, vA], 1)
        siB = vB
        for lp in params["layers"]:
            table = _node_pre(siA, siB, lp["n_encode"])
            gd, gs = _sc_gather(table, dst, src)
            msgs = _pair_gvp(gd, gs, lp["m_gvp"], 5000, split_out=False)
            agg = _sc_scatter(msgs, dst, zeros, n)
            siA, siB = _pair_gvp(table, agg, lp["u_gvp"], 5000,
                                 split_out=True)
        wd, sh, vhA, vhB, lpart = _head(siA, siB, wdfp[:, i], s[:, i],
                                        vt[:, i], params)
        s_h, wdfp_h, vA, vB = sh, wd, vhA, vhB
        out_labels.append(wd)
        out_s.append(sh)
        v48 = jnp.concatenate([vhA, vhB], 1)
        out_v.append(jnp.swapaxes(v48.reshape(n, 3, 16), 1, 2))
        lparts.append(lpart[0])

    label_losses = sum(p[0] for p in lparts) / ng
    feat_losses = sum((p[1] / (ng * 16) + p[2] / (ng * 48)) / 2.0
                      for p in lparts)
    loss = (label_losses + feat_losses) / t
    return (jnp.stack(out_labels, 1), jnp.stack(out_s, 1),
            jnp.stack(out_v, 1), loss)


# speedup vs baseline: 21.0534x; 1.0323x over previous
import functools

import jax
import jax.numpy as jnp
from jax import lax
from jax.experimental import pallas as pl
from jax.experimental.pallas import tpu as pltpu, tpu_sc as plsc

EPS = 1e-8


def _mm(a, b):
    return jax.lax.dot_general(a.astype(jnp.bfloat16), b.astype(jnp.bfloat16),
                               (((a.ndim - 1,), (0,)), ((), ())),
                               preferred_element_type=jnp.float32)

def _bd3(w):
    z = jnp.zeros_like(w)
    return jnp.concatenate([
        jnp.concatenate([w, z, z], 1),
        jnp.concatenate([z, w, z], 1),
        jnp.concatenate([z, z, w], 1)], 0)


def _norm3(a, b, c):
    return jnp.sqrt(jnp.clip(a * a + b * b + c * c, EPS, None))


def _node_pre_body(siA_ref, siB_ref, wh_ref, wv_ref, ws_ref, bs_ref, out_ref):
    siA = siA_ref[...]
    si = siA[:, :32]
    v48 = jnp.concatenate([siA[:, 32:40], siB_ref[...]], 1)
    hv = _mm(v48, _bd3(wh_ref[...]))
    vh = [hv[:, 16 * a:16 * a + 16] for a in range(3)]
    ov = _mm(hv, _bd3(wv_ref[...]))
    vo = [ov[:, 16 * a:16 * a + 16] for a in range(3)]
    vn = _norm3(*vh)
    se = _mm(jnp.concatenate([si, vn], 1), ws_ref[...]) + bs_ref[...]
    se = jnp.maximum(se, 0.0)
    sc = jax.nn.sigmoid(_norm3(*vo))
    pad = jnp.zeros((se.shape[0], 48), jnp.float32)
    out_ref[...] = jnp.concatenate(
        [se, vo[0] * sc, vo[1] * sc, vo[2] * sc, pad], 1)


def _pair_gvp_body(split_out, xd_ref, xs_ref,
                   wh_ref, wv_ref, ws_ref, bs_ref, *outs):
    xd, xs = xd_ref[...], xs_ref[...]
    vd = xd[:, 32:80]
    vs = xs[:, 32:80]
    vcat = jnp.concatenate(
        [jnp.concatenate([vd[:, 16 * a:16 * a + 16],
                          vs[:, 16 * a:16 * a + 16]], 1) for a in range(3)], 1)
    hv = _mm(vcat, _bd3(wh_ref[...]))
    vh = [hv[:, 32 * a:32 * a + 32] for a in range(3)]
    ov = _mm(hv, _bd3(wv_ref[...]))
    vo = [ov[:, 16 * a:16 * a + 16] for a in range(3)]
    vn = _norm3(*vh)
    sm = _mm(jnp.concatenate([xd[:, :32], xs[:, :32], vn], 1), ws_ref[...]) \
        + bs_ref[...]
    sm = jnp.maximum(sm, 0.0)
    sc = jax.nn.sigmoid(_norm3(*vo))
    vm = [vo[a] * sc for a in range(3)]
    if split_out:
        o1_ref, o2_ref = outs
        o1_ref[...] = jnp.concatenate([sm, vm[0][:, :8]], 1)
        o2_ref[...] = jnp.concatenate([vm[0][:, 8:], vm[1], vm[2]], 1)
    else:
        pad = jnp.zeros((sm.shape[0], 48), jnp.float32)
        outs[0][...] = jnp.concatenate([sm, vm[0], vm[1], vm[2], pad], 1)


def _head_body(stA_ref, stB_ref, wdfp_ref, s_ref, v_ref,
               whl_ref, wsl_ref, bsl_ref, lnw_ref, lnb_ref, whf_ref, wvf_ref,
               lab_ref, sh_ref, vhA_ref, vhB_ref, lp_ref):
    stA, stB = stA_ref[...], stB_ref[...]
    si = stA[:, :32]
    v48 = jnp.concatenate([stA[:, 32:40], stB], 1)
    hvl = _mm(v48, _bd3(whl_ref[...]))
    vnl = _norm3(hvl[:, :16], hvl[:, 16:32], hvl[:, 32:48])
    lab = _mm(jnp.concatenate([si, vnl], 1), wsl_ref[...]) + bsl_ref[...]
    lab = jnp.maximum(lab, 0.0)
    wd = _mm(lab, lnw_ref[...]) + lnb_ref[...]
    vfp = _mm(_mm(v48, _bd3(whf_ref[...])), _bd3(wvf_ref[...]))
    sh = _norm3(vfp[:, :16], vfp[:, 16:32], vfp[:, 32:48])
    sh3 = jnp.concatenate([sh, sh, sh], 1)
    vh = vfp / sh3
    lab_ref[...] = wd
    sh_ref[...] = sh
    vhA_ref[...] = vh[:, :8]
    vhB_ref[...] = vh[:, 8:]
    lt = jnp.sum(jnp.abs(wd - wdfp_ref[...]))
    st = jnp.sum(jnp.exp(sh - 2.0) * jnp.abs(sh - s_ref[...]))
    vt = jnp.sum(jnp.exp(vh - 2.0) * jnp.abs(vh - v_ref[...]))
    part = jnp.concatenate([lt.reshape(1, 1), st.reshape(1, 1),
                            vt.reshape(1, 1)], 1)

    @pl.when(pl.program_id(0) == 0)
    def _():
        lp_ref[...] = jnp.zeros_like(lp_ref)

    lp_ref[...] += part


def _full(shape2):
    return pl.BlockSpec(shape2, lambda i: tuple(0 for _ in shape2))


def _node_pre(siA, siB, p):
    n = siA.shape[0]
    bn = 5000 if n % 5000 == 0 else n
    return pl.pallas_call(
        _node_pre_body,
        grid=(n // bn,),
        in_specs=[pl.BlockSpec((bn, 40), lambda i: (i, 0)),
                  pl.BlockSpec((bn, 40), lambda i: (i, 0)),
                  _full((16, 16)), _full((16, 16)), _full((48, 32)),
                  _full((1, 32))],
        out_specs=pl.BlockSpec((bn, 128), lambda i: (i, 0)),
        out_shape=jax.ShapeDtypeStruct((n, 128), jnp.float32),
    )(siA, siB, p["wh"], p["wv"], p["ws"], p["bs"].reshape(1, 32))


def _pair_gvp(xd, xs, p, blk, split_out):
    n = xd.shape[0]
    bn = blk if n % blk == 0 else n
    if split_out:
        out_specs = [pl.BlockSpec((bn, 40), lambda i: (i, 0)),
                     pl.BlockSpec((bn, 40), lambda i: (i, 0))]
        out_shape = [jax.ShapeDtypeStruct((n, 40), jnp.float32),
                     jax.ShapeDtypeStruct((n, 40), jnp.float32)]
    else:
        out_specs = pl.BlockSpec((bn, 128), lambda i: (i, 0))
        out_shape = jax.ShapeDtypeStruct((n, 128), jnp.float32)
    return pl.pallas_call(
        functools.partial(_pair_gvp_body, split_out),
        grid=(n // bn,),
        in_specs=[pl.BlockSpec((bn, 128), lambda i: (i, 0)),
                  pl.BlockSpec((bn, 128), lambda i: (i, 0)),
                  _full((32, 32)), _full((32, 16)), _full((96, 32)),
                  _full((1, 32))],
        out_specs=out_specs,
        out_shape=out_shape,
    )(xd, xs, p["wh"], p["wv"], p["ws"], p["bs"].reshape(1, 32))


def _head(stA, stB, wdfp_i, s_i, vt_i, params):
    n = stA.shape[0]
    bn = 2000 if n % 2000 == 0 else n
    pla = params["label_gvp"]
    pf = params["feat_pred"]
    return pl.pallas_call(
        _head_body,
        grid=(n // bn,),
        in_specs=[pl.BlockSpec((bn, 40), lambda i: (i, 0)),
                  pl.BlockSpec((bn, 40), lambda i: (i, 0)),
                  pl.BlockSpec((bn, 1), lambda i: (i, 0)),
                  pl.BlockSpec((bn, 16), lambda i: (i, 0)),
                  pl.BlockSpec((bn, 48), lambda i: (i, 0)),
                  _full((16, 16)), _full((48, 64)), _full((1, 64)),
                  _full((64, 1)), _full((1, 1)),
                  _full((16, 16)), _full((16, 16))],
        out_specs=[pl.BlockSpec((bn, 1), lambda i: (i, 0)),
                   pl.BlockSpec((bn, 16), lambda i: (i, 0)),
                   pl.BlockSpec((bn, 8), lambda i: (i, 0)),
                   pl.BlockSpec((bn, 40), lambda i: (i, 0)),
                   pl.BlockSpec((1, 3), lambda i: (0, 0))],
        out_shape=[jax.ShapeDtypeStruct((n, 1), jnp.float32),
                   jax.ShapeDtypeStruct((n, 16), jnp.float32),
                   jax.ShapeDtypeStruct((n, 8), jnp.float32),
                   jax.ShapeDtypeStruct((n, 40), jnp.float32),
                   jax.ShapeDtypeStruct((1, 3), jnp.float32)],
    )(stA, stB, wdfp_i, s_i, vt_i,
      pla["wh"], pla["ws"], pla["bs"].reshape(1, 64),
      params["ln_w"], params["ln_b"].reshape(1, 1),
      pf["wh"], pf["wv"])


@functools.cache
def _sc_mesh():
    return plsc.VectorSubcoreMesh(core_axis_name="c", subcore_axis_name="s")


_CDG = 256
_CDS = 128


def _sc_gather(table, dst, src):
    e = dst.shape[0]
    per_w = e // 32
    n_main, tail = per_w // _CDG, per_w % _CDG

    @functools.partial(
        pl.kernel,
        out_type=(jax.ShapeDtypeStruct((e, 128), jnp.float32),
                  jax.ShapeDtypeStruct((e, 128), jnp.float32)),
        mesh=_sc_mesh(),
        scratch_types=[pltpu.VMEM((_CDG,), jnp.int32),
                       pltpu.VMEM((_CDG,), jnp.int32),
                       pltpu.VMEM((_CDG, 128), jnp.float32),
                       pltpu.VMEM((_CDG, 128), jnp.float32),
                       pltpu.SemaphoreType.DMA,
                       pltpu.SemaphoreType.DMA],
    )
    def k(tab, d_h, s_h, outd, outs, idxd, idxs, rowd, rows_, semd, sems):
        wid = lax.axis_index("s") * 2 + lax.axis_index("c")
        base = wid * per_w

        def chunk(off, sz):
            sl = pl.ds(0, sz)
            pltpu.sync_copy(d_h.at[pl.ds(off, sz)], idxd.at[sl])
            pltpu.sync_copy(s_h.at[pl.ds(off, sz)], idxs.at[sl])
            cd = pltpu.async_copy(tab.at[idxd.at[sl]], rowd.at[sl], semd)
            cs = pltpu.async_copy(tab.at[idxs.at[sl]], rows_.at[sl], sems)
            cd.wait()
            cs.wait()
            pltpu.sync_copy(rowd.at[sl], outd.at[pl.ds(off, sz)])
            pltpu.sync_copy(rows_.at[sl], outs.at[pl.ds(off, sz)])

        def body(j, carry):
            chunk(base + j * _CDG, _CDG)
            return carry

        lax.fori_loop(0, n_main, body, 0)
        if tail:
            chunk(base + n_main * _CDG, tail)

    return k(table, dst, src)


_R = 12800
_G = 128
_AR = _R + _G
_ZN = _AR // 16


def _sc_scatter(msgs, dst, zeros, n):
    e = msgs.shape[0]
    per_t = e // 16
    n_main, tail = per_t // _CDS, per_t % _CDS
    dn = 800
    dn3, dl3 = 728, n - 3 * _R - 15 * 728

    @functools.partial(
        pl.kernel,
        out_type=jax.ShapeDtypeStruct((n, 128), jnp.float32),
        mesh=_sc_mesh(),
        scratch_types=[pltpu.VMEM_SHARED((_AR, 128), jnp.float32),
                       pltpu.VMEM((_CDS,), jnp.int32),
                       pltpu.VMEM((_CDS,), jnp.int32),
                       pltpu.VMEM((_CDS, 128), jnp.float32)] +
                      ([pltpu.VMEM((tail,), jnp.int32),
                        pltpu.VMEM((tail,), jnp.int32),
                        pltpu.VMEM((tail, 128), jnp.float32)] if tail else []),
    )
    def k(m_h, d_h, z_h, out, acc, idxv, lidx, valv, *tailbufs):
        c = lax.axis_index("c")
        s = lax.axis_index("s")
        lanes = lax.iota(jnp.int32, 16)

        for phase in range(2):
            rng = 2 * c + phase
            base = rng * _R
            pltpu.sync_copy(z_h.at[pl.ds(0, _ZN)],
                            acc.at[pl.ds(s * _ZN, _ZN)])
            plsc.subcore_barrier()

            def chunk(j, off, sz, ibuf, lbuf, vbuf):
                garb = _R + ((lanes + j * 16) & (_G - 1))
                pltpu.sync_copy(d_h.at[pl.ds(off, sz)], ibuf)
                for i in range(sz // 16):
                    d16 = ibuf[pl.ds(16 * i, 16)]
                    loc = d16 - base
                    ok = (loc >= 0) & (loc < _R)
                    lbuf[pl.ds(16 * i, 16)] = jnp.where(ok, loc, garb)
                pltpu.sync_copy(m_h.at[pl.ds(off, sz)], vbuf)
                pltpu.sync_copy(vbuf, acc.at[lbuf], add=True)

            def body(j, carry):
                chunk(j, s * per_t + j * _CDS, _CDS, idxv, lidx, valv)
                return carry

            lax.fori_loop(0, n_main, body, 0)
            if tail:
                chunk(n_main, s * per_t + n_main * _CDS, tail, *tailbufs)
            plsc.subcore_barrier()

            @pl.when(rng < 3)
            def _():
                pltpu.sync_copy(acc.at[pl.ds(s * dn, dn)],
                                out.at[pl.ds(base + s * dn, dn)])

            @pl.when((rng == 3) & (s < 15))
            def _():
                pltpu.sync_copy(acc.at[pl.ds(s * dn3, dn3)],
                                out.at[pl.ds(base + s * dn3, dn3)])

            @pl.when((rng == 3) & (s == 15))
            def _():
                pltpu.sync_copy(acc.at[pl.ds(15 * dn3, dl3)],
                                out.at[pl.ds(base + 15 * dn3, dl3)])

            plsc.subcore_barrier()

    return k(msgs, dst, zeros)


def kernel(edge_index, s_fixed, s, v, wdfp, s_batch, params):
    n, t = s.shape[0], s.shape[1] - 1
    ng = 16
    src = edge_index[0]
    dst = edge_index[1]
    vt = jnp.swapaxes(v, -1, -2).reshape(n, t + 1, 48)
    zeros = jnp.zeros((_ZN, 128), jnp.float32)

    s_h = s[:, 0]
    wdfp_h = wdfp[:, 0]
    vA = vt[:, 0, :8]
    vB = vt[:, 0, 8:]

    out_labels, out_s, out_v, lparts = [], [], [], []
    for i in range(1, t + 1):
        siA = jnp.concatenate([s_fixed, s_h, wdfp_h, vA], 1)
        siB = vB
        for lp in params["layers"]:
            table = _node_pre(siA, siB, lp["n_encode"])
            gd, gs = _sc_gather(table, dst, src)
            msgs = _pair_gvp(gd, gs, lp["m_gvp"], 5000, split_out=False)
            agg = _sc_scatter(msgs, dst, zeros, n)
            siA, siB = _pair_gvp(table, agg, lp["u_gvp"], 5000,
                                 split_out=True)
        wd, sh, vhA, vhB, lpart = _head(siA, siB, wdfp[:, i], s[:, i],
                                        vt[:, i], params)
        s_h, wdfp_h, vA, vB = sh, wd, vhA, vhB
        out_labels.append(wd)
        out_s.append(sh)
        v48 = jnp.concatenate([vhA, vhB], 1)
        out_v.append(jnp.swapaxes(v48.reshape(n, 3, 16), 1, 2))
        lparts.append(lpart[0])

    label_losses = sum(p[0] for p in lparts) / ng
    feat_losses = sum((p[1] / (ng * 16) + p[2] / (ng * 48)) / 2.0
                      for p in lparts)
    loss = (label_losses + feat_losses) / t
    return (jnp.stack(out_labels, 1), jnp.stack(out_s, 1),
            jnp.stack(out_v, 1), loss)
